# baseline (speedup 1.0000x reference)
import jax
import jax.numpy as jnp
from jax import lax
from jax.experimental import pallas as pl
from jax.experimental.pallas import tpu as pltpu

B = 1024
T = 4096
FFN = 2048
HID = 8192

HBLK = 512
NH = HID // HBLK
FBLK = 128
NF = FFN // FBLK
GCHUNK = 256


def _body(idx_ref, hs_ref, w1_ref, w3_ref, w2_ref, rw_ref, o_ref,
          x_ref, g_ref, sem):
    j = pl.program_id(0)

    @pl.when(j == 0)
    def _gather():
        def issue(i, _):
            pltpu.make_async_copy(hs_ref.at[idx_ref[i]], x_ref.at[i], sem).start()
            return 0

        def drain(i, _):
            pltpu.make_async_copy(hs_ref.at[0], x_ref.at[0], sem).wait()
            return 0

        def per_chunk(c, _):
            lax.fori_loop(c * GCHUNK, (c + 1) * GCHUNK, issue, 0)
            lax.fori_loop(0, GCHUNK, drain, 0)
            return 0

        lax.fori_loop(0, B // GCHUNK, per_chunk, 0)

    @pl.when(j < NH)
    def _gate():
        x = x_ref[...].astype(jnp.bfloat16)
        w1 = w1_ref[...].astype(jnp.bfloat16)
        w3 = w3_ref[...].astype(jnp.bfloat16)
        h1 = lax.dot_general(x, w1, (((1,), (1,)), ((), ())),
                             preferred_element_type=jnp.float32)
        h3 = lax.dot_general(x, w3, (((1,), (1,)), ((), ())),
                             preferred_element_type=jnp.float32)
        g = (h1 * jax.nn.sigmoid(h1)) * h3 * rw_ref[...]
        g_ref[:, pl.ds(j * HBLK, HBLK)] = g.astype(jnp.bfloat16)

    @pl.when(j >= NH)
    def _down():
        w2 = w2_ref[...].astype(jnp.bfloat16)
        o_ref[...] = lax.dot_general(g_ref[...], w2, (((1,), (1,)), ((), ())),
                                     preferred_element_type=jnp.float32)


def kernel(hidden_states, input_idx, routing_weights, W1, W2, W3):
    return pl.pallas_call(
        _body,
        grid=(NH + NF,),
        in_specs=[
            pl.BlockSpec(memory_space=pltpu.SMEM),
            pl.BlockSpec(memory_space=pl.ANY),
            pl.BlockSpec((HBLK, FFN), lambda j: (jnp.minimum(j, NH - 1), 0)),
            pl.BlockSpec((HBLK, FFN), lambda j: (jnp.minimum(j, NH - 1), 0)),
            pl.BlockSpec((FBLK, HID), lambda j: (jnp.maximum(j - NH, 0), 0)),
            pl.BlockSpec((B, 1), lambda j: (0, 0)),
        ],
        out_specs=pl.BlockSpec((B, FBLK), lambda j: (0, jnp.maximum(j - NH, 0))),
        out_shape=jax.ShapeDtypeStruct((B, FFN), jnp.float32),
        scratch_shapes=[
            pltpu.VMEM((B, FFN), jnp.float32),
            pltpu.VMEM((B, HID), jnp.bfloat16),
            pltpu.SemaphoreType.DMA,
        ],
        compiler_params=pltpu.CompilerParams(
            dimension_semantics=("arbitrary",)),
    )(input_idx, hidden_states, W1, W3, W2, routing_weights)


# speedup vs baseline: 1.2693x; 1.2693x over previous
import jax
import jax.numpy as jnp
from jax import lax
from jax.experimental import pallas as pl
from jax.experimental.pallas import tpu as pltpu

B = 1024
T = 4096
FFN = 2048
HID = 8192

HBLK = 256
NH = HID // HBLK
FBLK = 256
NF = FFN // FBLK
GCHUNK = 256


def _body(idx_ref, hs_ref, w1_ref, w3_ref, w2_ref, rw_ref, o_ref,
          x_ref, g_ref, sem):
    j = pl.program_id(0)

    @pl.when(j == 0)
    def _gather():
        def issue(i, _):
            pltpu.make_async_copy(hs_ref.at[idx_ref[i]], x_ref.at[i], sem).start()
            return 0

        def drain(i, _):
            pltpu.make_async_copy(hs_ref.at[0], x_ref.at[0], sem).wait()
            return 0

        for c in range(B // GCHUNK):
            lax.fori_loop(c * GCHUNK, (c + 1) * GCHUNK, issue, 0, unroll=8)
            lax.fori_loop(0, GCHUNK, drain, 0, unroll=8)

    @pl.when(j < NH)
    def _gate():
        x = x_ref[...].astype(jnp.bfloat16)
        w1 = w1_ref[...].astype(jnp.bfloat16)
        w3 = w3_ref[...].astype(jnp.bfloat16)
        h1 = lax.dot_general(x, w1, (((1,), (1,)), ((), ())),
                             preferred_element_type=jnp.float32)
        h3 = lax.dot_general(x, w3, (((1,), (1,)), ((), ())),
                             preferred_element_type=jnp.float32)
        g = (h1 * jax.nn.sigmoid(h1)) * h3 * rw_ref[...]
        g_ref[:, pl.ds(j * HBLK, HBLK)] = g.astype(jnp.bfloat16)

    @pl.when(j >= NH)
    def _down():
        w2 = w2_ref[...].astype(jnp.bfloat16)
        o_ref[...] = lax.dot_general(g_ref[...], w2, (((1,), (1,)), ((), ())),
                                     preferred_element_type=jnp.float32)


def kernel(hidden_states, input_idx, routing_weights, W1, W2, W3):
    return pl.pallas_call(
        _body,
        grid=(NH + NF,),
        in_specs=[
            pl.BlockSpec(memory_space=pltpu.SMEM),
            pl.BlockSpec(memory_space=pl.ANY),
            pl.BlockSpec((HBLK, FFN), lambda j: (jnp.minimum(j, NH - 1), 0)),
            pl.BlockSpec((HBLK, FFN), lambda j: (jnp.minimum(j, NH - 1), 0)),
            pl.BlockSpec((FBLK, HID), lambda j: (jnp.maximum(j - NH, 0), 0)),
            pl.BlockSpec((B, 1), lambda j: (0, 0)),
        ],
        out_specs=pl.BlockSpec((B, FBLK), lambda j: (0, jnp.maximum(j - NH, 0))),
        out_shape=jax.ShapeDtypeStruct((B, FFN), jnp.float32),
        scratch_shapes=[
            pltpu.VMEM((B, FFN), jnp.float32),
            pltpu.VMEM((B, HID), jnp.bfloat16),
            pltpu.SemaphoreType.DMA,
        ],
        compiler_params=pltpu.CompilerParams(
            dimension_semantics=("arbitrary",)),
    )(input_idx, hidden_states, W1, W3, W2, routing_weights)


# speedup vs baseline: 1.2914x; 1.0174x over previous
import jax
import jax.numpy as jnp
from jax import lax
from jax.experimental import pallas as pl
from jax.experimental.pallas import tpu as pltpu

B = 1024
T = 4096
FFN = 2048
HID = 8192

HBLK = 256
NH = HID // HBLK
FBLK = 256
NF = FFN // FBLK
GCHUNK = 256


def _body(idx_ref, hs_ref, w1_ref, w3_ref, w2_ref, rw_ref, o_ref,
          x_ref, g_ref, sem):
    j = pl.program_id(0)

    @pl.when(j == 0)
    def _gather():
        def issue(i, _):
            pltpu.make_async_copy(hs_ref.at[idx_ref[i]], x_ref.at[i], sem).start()
            return 0

        lax.fori_loop(0, B, issue, 0, unroll=8)
        pltpu.make_async_copy(hs_ref.at[pl.ds(0, B)], x_ref, sem).wait()

    @pl.when(j < NH)
    def _gate():
        x = x_ref[...].astype(jnp.bfloat16)
        w1 = w1_ref[...].astype(jnp.bfloat16)
        w3 = w3_ref[...].astype(jnp.bfloat16)
        h1 = lax.dot_general(x, w1, (((1,), (1,)), ((), ())),
                             preferred_element_type=jnp.float32)
        h3 = lax.dot_general(x, w3, (((1,), (1,)), ((), ())),
                             preferred_element_type=jnp.float32)
        g = (h1 * jax.nn.sigmoid(h1)) * h3 * rw_ref[...]
        g_ref[:, pl.ds(j * HBLK, HBLK)] = g.astype(jnp.bfloat16)

    @pl.when(j >= NH)
    def _down():
        w2 = w2_ref[...].astype(jnp.bfloat16)
        o_ref[...] = lax.dot_general(g_ref[...], w2, (((1,), (1,)), ((), ())),
                                     preferred_element_type=jnp.float32)


def kernel(hidden_states, input_idx, routing_weights, W1, W2, W3):
    return pl.pallas_call(
        _body,
        grid=(NH + NF,),
        in_specs=[
            pl.BlockSpec(memory_space=pltpu.SMEM),
            pl.BlockSpec(memory_space=pl.ANY),
            pl.BlockSpec((HBLK, FFN), lambda j: (jnp.minimum(j, NH - 1), 0)),
            pl.BlockSpec((HBLK, FFN), lambda j: (jnp.minimum(j, NH - 1), 0)),
            pl.BlockSpec((FBLK, HID), lambda j: (jnp.maximum(j - NH, 0), 0)),
            pl.BlockSpec((B, 1), lambda j: (0, 0)),
        ],
        out_specs=pl.BlockSpec((B, FBLK), lambda j: (0, jnp.maximum(j - NH, 0))),
        out_shape=jax.ShapeDtypeStruct((B, FFN), jnp.float32),
        scratch_shapes=[
            pltpu.VMEM((B, FFN), jnp.float32),
            pltpu.VMEM((B, HID), jnp.bfloat16),
            pltpu.SemaphoreType.DMA,
        ],
        compiler_params=pltpu.CompilerParams(
            dimension_semantics=("arbitrary",)),
    )(input_idx, hidden_states, W1, W3, W2, routing_weights)


# speedup vs baseline: 1.2940x; 1.0020x over previous
import jax
import jax.numpy as jnp
from jax import lax
from jax.experimental import pallas as pl
from jax.experimental.pallas import tpu as pltpu

B = 1024
T = 4096
FFN = 2048
HID = 8192

HBLK = 256
NH = HID // HBLK
FBLK = 256
NF = FFN // FBLK
GCHUNK = 256


def _body(idx_ref, hs_ref, w1_ref, w3_ref, w2_ref, rw_ref, o_ref,
          x_ref, g_ref, sem):
    j = pl.program_id(0)

    @pl.when(j == 0)
    def _gather():
        def issue(i, _):
            pltpu.make_async_copy(hs_ref.at[idx_ref[i]], x_ref.at[i], sem).start()
            return 0

        lax.fori_loop(0, B, issue, 0, unroll=16)
        pltpu.make_async_copy(hs_ref.at[pl.ds(0, B)], x_ref, sem).wait()

    @pl.when(j < NH)
    def _gate():
        x = x_ref[...].astype(jnp.bfloat16)
        w1 = w1_ref[...].astype(jnp.bfloat16)
        w3 = w3_ref[...].astype(jnp.bfloat16)
        h1 = lax.dot_general(x, w1, (((1,), (1,)), ((), ())),
                             preferred_element_type=jnp.float32)
        h3 = lax.dot_general(x, w3, (((1,), (1,)), ((), ())),
                             preferred_element_type=jnp.float32)
        g = (h1 * jax.nn.sigmoid(h1)) * h3 * rw_ref[...]
        g_ref[:, pl.ds(j * HBLK, HBLK)] = g.astype(jnp.bfloat16)

    @pl.when(j >= NH)
    def _down():
        w2 = w2_ref[...].astype(jnp.bfloat16)
        o_ref[...] = lax.dot_general(g_ref[...], w2, (((1,), (1,)), ((), ())),
                                     preferred_element_type=jnp.float32)


def kernel(hidden_states, input_idx, routing_weights, W1, W2, W3):
    return pl.pallas_call(
        _body,
        grid=(NH + NF,),
        in_specs=[
            pl.BlockSpec(memory_space=pltpu.SMEM),
            pl.BlockSpec(memory_space=pl.ANY),
            pl.BlockSpec((HBLK, FFN), lambda j: (jnp.minimum(j, NH - 1), 0)),
            pl.BlockSpec((HBLK, FFN), lambda j: (jnp.minimum(j, NH - 1), 0)),
            pl.BlockSpec((FBLK, HID), lambda j: (jnp.maximum(j - NH, 0), 0)),
            pl.BlockSpec((B, 1), lambda j: (0, 0)),
        ],
        out_specs=pl.BlockSpec((B, FBLK), lambda j: (0, jnp.maximum(j - NH, 0))),
        out_shape=jax.ShapeDtypeStruct((B, FFN), jnp.float32),
        scratch_shapes=[
            pltpu.VMEM((B, FFN), jnp.float32),
            pltpu.VMEM((B, HID), jnp.bfloat16),
            pltpu.SemaphoreType.DMA,
        ],
        compiler_params=pltpu.CompilerParams(
            dimension_semantics=("arbitrary",)),
    )(input_idx, hidden_states, W1, W3, W2, routing_weights)


# speedup vs baseline: 1.3039x; 1.0076x over previous
import jax
import jax.numpy as jnp
from jax import lax
from jax.experimental import pallas as pl
from jax.experimental.pallas import tpu as pltpu

B = 1024
T = 4096
FFN = 2048
HID = 8192

HBLK = 256
NH = HID // HBLK
FBLK = 256
NF = FFN // FBLK
GCHUNK = 256


def _body(idx_ref, hs_ref, w1_ref, w3_ref, w2_ref, rw_ref, o_ref,
          x_ref, g_ref, sem, sem2):
    j = pl.program_id(0)

    def _gate_rows(lo, nrows):
        x = x_ref[pl.ds(lo, nrows), :].astype(jnp.bfloat16)
        w1 = w1_ref[...].astype(jnp.bfloat16)
        w3 = w3_ref[...].astype(jnp.bfloat16)
        h1 = lax.dot_general(x, w1, (((1,), (1,)), ((), ())),
                             preferred_element_type=jnp.float32)
        h3 = lax.dot_general(x, w3, (((1,), (1,)), ((), ())),
                             preferred_element_type=jnp.float32)
        g = (h1 * jax.nn.sigmoid(h1)) * h3 * rw_ref[pl.ds(lo, nrows), :]
        g_ref[pl.ds(lo, nrows), pl.ds(j * HBLK, HBLK)] = g.astype(jnp.bfloat16)

    @pl.when(j == 0)
    def _gather_and_first_gate():
        half = B // 2

        def issue_a(i, _):
            pltpu.make_async_copy(hs_ref.at[idx_ref[i]], x_ref.at[i], sem).start()
            return 0

        def issue_b(i, _):
            pltpu.make_async_copy(hs_ref.at[idx_ref[i]], x_ref.at[i], sem2).start()
            return 0

        lax.fori_loop(0, half, issue_a, 0, unroll=16)
        lax.fori_loop(half, B, issue_b, 0, unroll=16)
        pltpu.make_async_copy(hs_ref.at[pl.ds(0, half)],
                              x_ref.at[pl.ds(0, half)], sem).wait()
        _gate_rows(0, half)
        pltpu.make_async_copy(hs_ref.at[pl.ds(0, half)],
                              x_ref.at[pl.ds(half, half)], sem2).wait()
        _gate_rows(half, half)

    @pl.when((j > 0) & (j < NH))
    def _gate():
        _gate_rows(0, B)

    @pl.when(j >= NH)
    def _down():
        w2 = w2_ref[...].astype(jnp.bfloat16)
        o_ref[...] = lax.dot_general(g_ref[...], w2, (((1,), (1,)), ((), ())),
                                     preferred_element_type=jnp.float32)


def kernel(hidden_states, input_idx, routing_weights, W1, W2, W3):
    return pl.pallas_call(
        _body,
        grid=(NH + NF,),
        in_specs=[
            pl.BlockSpec(memory_space=pltpu.SMEM),
            pl.BlockSpec(memory_space=pl.ANY),
            pl.BlockSpec((HBLK, FFN), lambda j: (jnp.minimum(j, NH - 1), 0)),
            pl.BlockSpec((HBLK, FFN), lambda j: (jnp.minimum(j, NH - 1), 0)),
            pl.BlockSpec((FBLK, HID), lambda j: (jnp.maximum(j - NH, 0), 0)),
            pl.BlockSpec((B, 1), lambda j: (0, 0)),
        ],
        out_specs=pl.BlockSpec((B, FBLK), lambda j: (0, jnp.maximum(j - NH, 0))),
        out_shape=jax.ShapeDtypeStruct((B, FFN), jnp.float32),
        scratch_shapes=[
            pltpu.VMEM((B, FFN), jnp.float32),
            pltpu.VMEM((B, HID), jnp.bfloat16),
            pltpu.SemaphoreType.DMA,
            pltpu.SemaphoreType.DMA,
        ],
        compiler_params=pltpu.CompilerParams(
            dimension_semantics=("arbitrary",)),
    )(input_idx, hidden_states, W1, W3, W2, routing_weights)
